# baseline (speedup 1.0000x reference)
import functools

import jax
import jax.numpy as jnp
from jax import lax
from jax.experimental import pallas as pl
from jax.experimental.pallas import tpu as pltpu
from jax.experimental.pallas import tpu_sc as plsc

B = 16384
D = 128
_info = plsc.get_sparse_core_info()
NC, NS, L = _info.num_cores, _info.num_subcores, _info.num_lanes
NW = NC * NS
PW = B // NW
NCH = PW // L
CH = 32
CAP = PW + 4 * CH

assert B % (8 * NW) == 0 and PW % L == 0

_mesh = plsc.VectorSubcoreMesh(core_axis_name="c", subcore_axis_name="s")


@functools.partial(
    pl.kernel,
    out_type=jax.ShapeDtypeStruct((B,), jnp.float32),
    mesh=_mesh,
    scratch_types=[
        pltpu.VMEM((PW,), jnp.int32),
        pltpu.VMEM((PW,), jnp.int32),
        pltpu.VMEM((PW,), jnp.int32),
        pltpu.VMEM((PW,), jnp.int32),
        pltpu.VMEM((CAP,), jnp.int32),
        pltpu.VMEM((CAP,), jnp.int32),
        pltpu.VMEM((CAP,), jnp.int32),
        pltpu.VMEM((CAP,), jnp.int32),
        *[pltpu.VMEM((CH, D), jnp.float32) for _ in range(6)],
        pltpu.VMEM((L, L), jnp.float32),
        pltpu.VMEM((PW,), jnp.float32),
        *[pltpu.SemaphoreType.DMA for _ in range(6)],
    ],
    compiler_params=pltpu.CompilerParams(needs_layout_passes=False),
)
def _mergl_sc(i_h, j_h, k_h, lab_h,
              ue0, ie0, te0, ue1, ie1, te1, ue2, ie2, te2, ue3, ie3, te3,
              out_h,
              iv, jv, kv, lv, ci, cj, ck, cpos,
              ub0, vb0, tb0, ub1, vb1, tb1, pb, ob,
              s0, s1, s2, s3, s4, s5):
    bufs = ((ub0, vb0, tb0, s0, s1, s2), (ub1, vb1, tb1, s3, s4, s5))
    tables = ((ue0, ie0, te0), (ue1, ie1, te1),
              (ue2, ie2, te2), (ue3, ie3, te3))
    wid = lax.axis_index("s") * NC + lax.axis_index("c")
    base = wid * PW
    lane = lax.iota(jnp.int32, L)
    ones = jnp.ones((L,), jnp.int32)
    zeros = jnp.zeros((L,), jnp.int32)
    zero = jnp.int32(0)
    zf = jnp.zeros((L,), jnp.float32)

    pltpu.sync_copy(i_h.at[pl.ds(base, PW)], iv)
    pltpu.sync_copy(j_h.at[pl.ds(base, PW)], jv)
    pltpu.sync_copy(k_h.at[pl.ds(base, PW)], kv)
    pltpu.sync_copy(lab_h.at[pl.ds(base, PW)], lv)

    def cnt_body(c, cnts):
        l16 = lv[pl.ds(c * L, L)]
        new = []
        for lbl in range(4):
            mi = jnp.where(l16 == lbl, ones, zeros)
            new.append(cnts[lbl] + jnp.sum(mi))
        return tuple(new)

    n0, n1, n2, n3 = lax.fori_loop(0, NCH, cnt_body, (zero,) * 4)

    def ceilch(x):
        return (x + (CH - 1)) // CH * CH

    b1 = ceilch(n0)
    b2 = b1 + ceilch(n1)
    b3 = b2 + ceilch(n2)
    total = b3 + ceilch(n3)
    bases = (zero, b1, b2, b3)
    rends = (n0, b1 + n1, b2 + n2, b3 + n3)
    uppers = (b1, b2, b3, jnp.int32(CAP))

    def comp_body(c, cnts):
        off = c * L
        l16 = lv[pl.ds(off, L)]
        i16 = iv[pl.ds(off, L)]
        j16 = jv[pl.ds(off, L)]
        k16 = kv[pl.ds(off, L)]
        p16 = off + lane
        new = []
        for lbl in range(4):
            m = l16 == lbl
            mi = jnp.where(m, ones, zeros)
            dest = bases[lbl] + cnts[lbl] + plsc.cumsum(mi) - mi
            plsc.store_scatter(ci, [dest], i16, mask=m)
            plsc.store_scatter(cj, [dest], j16, mask=m)
            plsc.store_scatter(ck, [dest], k16, mask=m)
            plsc.store_scatter(cpos, [dest], p16, mask=m)
            new.append(cnts[lbl] + jnp.sum(mi))
        return tuple(new)

    lax.fori_loop(0, NCH, comp_body, (zero,) * 4)

    for lbl in range(4):
        for t in range(CH // L):
            pad = rends[lbl] + t * L + lane
            pm = pad < uppers[lbl]
            plsc.store_scatter(ci, [pad], zeros, mask=pm)
            plsc.store_scatter(cj, [pad], zeros, mask=pm)
            plsc.store_scatter(ck, [pad], zeros, mask=pm)

    def slot_lbl(off):
        return (jnp.where(off >= b1, 1, 0) + jnp.where(off >= b2, 1, 0)
                + jnp.where(off >= b3, 1, 0))

    def slot_rend(off):
        lbs = slot_lbl(off)
        r = jnp.where(lbs == 0, rends[0],
                      jnp.where(lbs == 1, rends[1],
                                jnp.where(lbs == 2, rends[2], rends[3])))
        return r

    def copies(s, off, tab):
        off = pl.multiple_of(off, CH)
        b = bufs[s]
        return (
            pltpu.make_async_copy(tab[0].at[ci.at[pl.ds(off, CH)]], b[0], b[3]),
            pltpu.make_async_copy(tab[1].at[cj.at[pl.ds(off, CH)]], b[1], b[4]),
            pltpu.make_async_copy(tab[2].at[ck.at[pl.ds(off, CH)]], b[2], b[5]),
        )

    def issue(s, off):
        lbs = slot_lbl(off)
        for l in range(4):
            @pl.when(lbs == l)
            def _(l=l):
                for c in copies(s, off, tables[l]):
                    c.start()

    def wait(s, off):
        for c in copies(s, off, tables[0]):
            c.wait()

    def compute(s, off):
        off = pl.multiple_of(off, CH)
        ub, vb, tb = bufs[s][0], bufs[s][1], bufs[s][2]
        rem = slot_rend(off) - off
        for h in range(CH // L):
            for e in range(L):
                r = h * L + e
                a = (ub[r, pl.ds(0, L)] * vb[r, pl.ds(0, L)]
                     * tb[r, pl.ds(0, L)])
                for q in range(1, D // L):
                    a = a + (ub[r, pl.ds(q * L, L)] * vb[r, pl.ds(q * L, L)]
                             * tb[r, pl.ds(q * L, L)])
                pb[e, pl.ds(0, L)] = a
            col = zeros
            r0 = zf
            r1 = zf
            for c in range(L):
                g = plsc.load_gather(pb, [lane, col])
                if c % 2 == 0:
                    r0 = r0 + g
                else:
                    r1 = r1 + g
                if c < L - 1:
                    col = col + ones
            res = r0 + r1
            ok = lane < (rem - h * L)
            p16 = cpos[pl.ds(off + h * L, L)]
            plsc.store_scatter(ob, [p16], res, mask=ok)

    issue(0, zero)

    def pair_body(off):
        off = pl.multiple_of(off, CH)
        nxt = off + CH
        nxt2 = off + 2 * CH

        @pl.when(nxt < total)
        def _():
            issue(1, nxt)

        wait(0, off)
        compute(0, off)

        @pl.when(nxt2 < total)
        def _():
            issue(0, nxt2)

        @pl.when(nxt < total)
        def _():
            wait(1, nxt)
            compute(1, nxt)

        return nxt2

    lax.while_loop(lambda off: off < total, pair_body, zero)

    pltpu.sync_copy(ob, out_h.at[pl.ds(base, PW)])


def kernel(i, j, k, labels,
           ue0, ie0, te0, ue1, ie1, te1, ue2, ie2, te2, ue3, ie3, te3):
    i = i.astype(jnp.int32)
    j = j.astype(jnp.int32)
    k = k.astype(jnp.int32)
    labels = labels.astype(jnp.int32)
    return _mergl_sc(i, j, k, labels,
                     ue0, ie0, te0, ue1, ie1, te1,
                     ue2, ie2, te2, ue3, ie3, te3)


# speedup vs baseline: 1.5442x; 1.5442x over previous
import functools

import jax
import jax.numpy as jnp
from jax import lax
from jax.experimental import pallas as pl
from jax.experimental.pallas import tpu as pltpu
from jax.experimental.pallas import tpu_sc as plsc

B = 16384
D = 128
_info = plsc.get_sparse_core_info()
NC, NS, L = _info.num_cores, _info.num_subcores, _info.num_lanes
NW = NC * NS
PW = B // NW
NCH = PW // L
CH = 16
CAP = PW + 4 * CH

assert B % (8 * NW) == 0 and PW % L == 0

_mesh = plsc.VectorSubcoreMesh(core_axis_name="c", subcore_axis_name="s")


@functools.partial(
    pl.kernel,
    out_type=jax.ShapeDtypeStruct((B,), jnp.float32),
    mesh=_mesh,
    scratch_types=[
        pltpu.VMEM((PW,), jnp.int32),
        pltpu.VMEM((PW,), jnp.int32),
        pltpu.VMEM((PW,), jnp.int32),
        pltpu.VMEM((PW,), jnp.int32),
        pltpu.VMEM((CAP,), jnp.int32),
        pltpu.VMEM((CAP,), jnp.int32),
        pltpu.VMEM((CAP,), jnp.int32),
        pltpu.VMEM((CAP,), jnp.int32),
        *[pltpu.VMEM((CH, D), jnp.float32) for _ in range(6)],
        pltpu.VMEM((L, L), jnp.float32),
        pltpu.VMEM((PW,), jnp.float32),
        *[pltpu.SemaphoreType.DMA for _ in range(6)],
    ],
    compiler_params=pltpu.CompilerParams(needs_layout_passes=False),
)
def _mergl_sc(i_h, j_h, k_h, lab_h,
              ue0, ie0, te0, ue1, ie1, te1, ue2, ie2, te2, ue3, ie3, te3,
              out_h,
              iv, jv, kv, lv, ci, cj, ck, cpos,
              ub0, vb0, tb0, ub1, vb1, tb1, pb, ob,
              s0, s1, s2, s3, s4, s5):
    bufs = ((ub0, vb0, tb0, s0, s1, s2), (ub1, vb1, tb1, s3, s4, s5))
    tables = ((ue0, ie0, te0), (ue1, ie1, te1),
              (ue2, ie2, te2), (ue3, ie3, te3))
    wid = lax.axis_index("s") * NC + lax.axis_index("c")
    base = wid * PW
    lane = lax.iota(jnp.int32, L)
    ones = jnp.ones((L,), jnp.int32)
    zeros = jnp.zeros((L,), jnp.int32)
    zero = jnp.int32(0)
    zf = jnp.zeros((L,), jnp.float32)

    pltpu.sync_copy(i_h.at[pl.ds(base, PW)], iv)
    pltpu.sync_copy(j_h.at[pl.ds(base, PW)], jv)
    pltpu.sync_copy(k_h.at[pl.ds(base, PW)], kv)
    pltpu.sync_copy(lab_h.at[pl.ds(base, PW)], lv)

    def cnt_body(c, cnts):
        l16 = lv[pl.ds(c * L, L)]
        new = []
        for lbl in range(4):
            mi = jnp.where(l16 == lbl, ones, zeros)
            new.append(cnts[lbl] + jnp.sum(mi))
        return tuple(new)

    n0, n1, n2, n3 = lax.fori_loop(0, NCH, cnt_body, (zero,) * 4)

    def ceil16(x):
        return (x + (L - 1)) // L * L

    b1 = ceil16(n0)
    b2 = b1 + ceil16(n1)
    b3 = b2 + ceil16(n2)
    total = b3 + ceil16(n3)
    bases = (zero, b1, b2, b3)
    rends = (n0, b1 + n1, b2 + n2, b3 + n3)
    uppers = (b1, b2, b3, jnp.int32(CAP))

    def comp_body(c, cnts):
        off = c * L
        l16 = lv[pl.ds(off, L)]
        i16 = iv[pl.ds(off, L)]
        j16 = jv[pl.ds(off, L)]
        k16 = kv[pl.ds(off, L)]
        p16 = off + lane
        new = []
        for lbl in range(4):
            m = l16 == lbl
            mi = jnp.where(m, ones, zeros)
            dest = bases[lbl] + cnts[lbl] + plsc.cumsum(mi) - mi
            plsc.store_scatter(ci, [dest], i16, mask=m)
            plsc.store_scatter(cj, [dest], j16, mask=m)
            plsc.store_scatter(ck, [dest], k16, mask=m)
            plsc.store_scatter(cpos, [dest], p16, mask=m)
            new.append(cnts[lbl] + jnp.sum(mi))
        return tuple(new)

    lax.fori_loop(0, NCH, comp_body, (zero,) * 4)

    for lbl in range(4):
        pad = rends[lbl] + lane
        pm = pad < uppers[lbl]
        plsc.store_scatter(ci, [pad], zeros, mask=pm)
        plsc.store_scatter(cj, [pad], zeros, mask=pm)
        plsc.store_scatter(ck, [pad], zeros, mask=pm)

    def slot_lbl(off):
        return (jnp.where(off >= b1, 1, 0) + jnp.where(off >= b2, 1, 0)
                + jnp.where(off >= b3, 1, 0))

    def slot_rend(off):
        lbs = slot_lbl(off)
        r = jnp.where(lbs == 0, rends[0],
                      jnp.where(lbs == 1, rends[1],
                                jnp.where(lbs == 2, rends[2], rends[3])))
        return r

    def copies(s, off, tab):
        off = pl.multiple_of(off, CH)
        b = bufs[s]
        return (
            pltpu.make_async_copy(tab[0].at[ci.at[pl.ds(off, CH)]], b[0], b[3]),
            pltpu.make_async_copy(tab[1].at[cj.at[pl.ds(off, CH)]], b[1], b[4]),
            pltpu.make_async_copy(tab[2].at[ck.at[pl.ds(off, CH)]], b[2], b[5]),
        )

    def issue(s, off):
        lbs = slot_lbl(off)
        for l in range(4):
            @pl.when(lbs == l)
            def _(l=l):
                for c in copies(s, off, tables[l]):
                    c.start()

    def wait(s, off):
        for c in copies(s, off, tables[0]):
            c.wait()

    def compute(s, off):
        off = pl.multiple_of(off, CH)
        ub, vb, tb = bufs[s][0], bufs[s][1], bufs[s][2]
        def elem_body(e):
            a = (ub[e, pl.ds(0, L)] * vb[e, pl.ds(0, L)] * tb[e, pl.ds(0, L)])
            for q in range(1, D // L):
                a = a + (ub[e, pl.ds(q * L, L)] * vb[e, pl.ds(q * L, L)]
                         * tb[e, pl.ds(q * L, L)])
            pb[e, pl.ds(0, L)] = a

        plsc.parallel_loop(0, L, unroll=4)(elem_body)
        col = zeros
        r0 = zf
        r1 = zf
        for c in range(L):
            g = plsc.load_gather(pb, [lane, col])
            if c % 2 == 0:
                r0 = r0 + g
            else:
                r1 = r1 + g
            if c < L - 1:
                col = col + ones
        res = r0 + r1
        rem = slot_rend(off) - off
        ok = lane < rem
        p16 = cpos[pl.ds(off, L)]
        plsc.store_scatter(ob, [p16], res, mask=ok)

    issue(0, zero)

    def pair_body(off):
        off = pl.multiple_of(off, CH)
        nxt = off + CH
        nxt2 = off + 2 * CH

        @pl.when(nxt < total)
        def _():
            issue(1, nxt)

        wait(0, off)
        compute(0, off)

        @pl.when(nxt2 < total)
        def _():
            issue(0, nxt2)

        @pl.when(nxt < total)
        def _():
            wait(1, nxt)
            compute(1, nxt)

        return nxt2

    lax.while_loop(lambda off: off < total, pair_body, zero)

    pltpu.sync_copy(ob, out_h.at[pl.ds(base, PW)])


def kernel(i, j, k, labels,
           ue0, ie0, te0, ue1, ie1, te1, ue2, ie2, te2, ue3, ie3, te3):
    i = i.astype(jnp.int32)
    j = j.astype(jnp.int32)
    k = k.astype(jnp.int32)
    labels = labels.astype(jnp.int32)
    return _mergl_sc(i, j, k, labels,
                     ue0, ie0, te0, ue1, ie1, te1,
                     ue2, ie2, te2, ue3, ie3, te3)
